# Initial kernel scaffold; baseline (speedup 1.0000x reference)
import jax
import jax.numpy as jnp
from jax.experimental import pallas as pl


def kernel(x, adj_row, adj_col, adj_val, sizes_subg, W0, b0, W1, b1, attention, scale, offset):
    raise NotImplementedError("write your pallas kernel here")


# speedup vs baseline: 35.5597x; 35.5597x over previous
import functools

import jax
import jax.numpy as jnp
from jax import lax
from jax.experimental import pallas as pl
from jax.experimental.pallas import tpu as pltpu
from jax.experimental.pallas import tpu_sc as plsc

N = 10000
E = 320000
DIM = 128
H = 4
D = 32

NC = 2
NS = 16
NW = NC * NS
NPW = 320
CH = 128
FW = 144
ACCW = 132
ACCLEN = NPW * ACCW
OUTLEN = NW * ACCLEN
NCHUNK = E // CH

_i32 = jnp.int32
_f32 = jnp.float32


def _dot(a, b):
    return jnp.dot(a, b, precision=lax.Precision.HIGHEST,
                   preferred_element_type=_f32)


def _pre_body(x_ref, w0t_ref, b0_ref, w1t_ref, b1_ref, a0_ref, a1_ref,
              fs_ref, fnext_ref, eas_ref):
    x = x_ref[...]
    fs = jnp.maximum(_dot(x, w0t_ref[...]) + b0_ref[...], 0.0)
    fn = jnp.maximum(_dot(x, w1t_ref[...]) + b1_ref[...], 0.0)
    att_s = _dot(fs, a0_ref[...])
    att_n = _dot(fn, a1_ref[...])
    att_s = jnp.where(att_s > 0, att_s, 0.2 * att_s)
    att_n = jnp.where(att_n > 0, att_n, 0.2 * att_n)
    c = jnp.max(att_s, axis=0) + jnp.max(att_n, axis=0)
    eas = jnp.exp(att_s - c[None, :])
    ean = jnp.exp(att_n)
    fs_ref[...] = fs
    fnext_ref[...] = jnp.concatenate(
        [fn, ean, jnp.zeros((N, FW - DIM - H), _f32)], axis=1)
    eas_ref[...] = jnp.concatenate(
        [eas, jnp.zeros((NW * NPW - N, H), _f32)], axis=0)


_pre_call = pl.pallas_call(
    _pre_body,
    out_shape=(
        jax.ShapeDtypeStruct((N, DIM), _f32),
        jax.ShapeDtypeStruct((N, FW), _f32),
        jax.ShapeDtypeStruct((NW * NPW, H), _f32),
    ),
)


def _sc_body(fnext_hbm, eas_hbm, row_hbm, col_hbm, val_hbm, bounds_hbm,
             out_hbm,
             acc_v, fnextc_v, rowc_v, colc_v, valc_v, easloc_v, wbuf_v,
             rlocbuf_v, bounds_v, sem):
    wid = lax.axis_index("s") * NC + lax.axis_index("c")
    nbase = wid * NPW
    nhi = jnp.minimum(nbase + NPW, N)

    pltpu.sync_copy(bounds_hbm, bounds_v)

    def _rdscalar(pos):
        v = plsc.load_gather(bounds_v, [jnp.full((16,), pos, _i32)])
        return jnp.max(v)

    c0 = _rdscalar(wid)
    c1 = _rdscalar(NW + wid)

    zz = jnp.zeros((16,), _f32)

    def _zero(i, carry):
        acc_v[pl.ds(i * 16, 16)] = zz
        return carry

    lax.fori_loop(0, ACCLEN // 16, _zero, 0)

    pltpu.sync_copy(eas_hbm.at[pl.ds(nbase * H, NPW * H)], easloc_v)

    iot = lax.iota(_i32, 16)

    def _chunk(g, carry):
        eb = g * CH
        pltpu.sync_copy(row_hbm.at[pl.ds(eb, CH)], rowc_v)
        pltpu.sync_copy(col_hbm.at[pl.ds(eb, CH)], colc_v)
        pltpu.sync_copy(val_hbm.at[pl.ds(eb, CH)], valc_v)
        pltpu.async_copy(fnext_hbm.at[colc_v], fnextc_v, sem).wait()

        for g8 in range(CH // 16):
            row16 = rowc_v[pl.ds(g8 * 16, 16)]
            val16 = valc_v[pl.ds(g8 * 16, 16)]
            m = (row16 >= nbase) & (row16 < nhi)
            rloc = jnp.clip(row16 - nbase, 0, NPW - 1)
            rlocbuf_v[pl.ds(g8 * 16, 16)] = rloc
            e16 = iot + g8 * 16
            for k in range(H):
                easv = plsc.load_gather(easloc_v, [rloc * H + k])
                anv = plsc.load_gather(
                    fnextc_v, [e16, jnp.full((16,), DIM + k, _i32)])
                w = jnp.where(m, easv * anv * val16, 0.0)
                wbuf_v[pl.ds(k * CH + g8 * 16, 16)] = w

        def _edges(t, carry):
            for i2 in range(8):
                e = t * 8 + i2
                ef = jnp.full((16,), 0, _i32) + e
                rb = plsc.load_gather(rlocbuf_v, [ef])
                rowaddr = rb * ACCW + iot
                wks = []
                for k in range(H):
                    wkb = plsc.load_gather(wbuf_v, [ef + k * CH])
                    wks.append(wkb)
                    for j in (2 * k, 2 * k + 1):
                        v = plsc.load_gather(
                            fnextc_v, [ef, iot + j * 16])
                        plsc.addupdate_scatter(
                            acc_v, [rowaddr + j * 16], v * wkb)
                wp = wks[0]
                wp = jnp.where(iot == 1, wks[1], wp)
                wp = jnp.where(iot == 2, wks[2], wp)
                wp = jnp.where(iot == 3, wks[3], wp)
                plsc.addupdate_scatter(
                    acc_v, [rb * ACCW + DIM + iot], wp, mask=iot < H)
            return carry

        lax.fori_loop(0, CH // 8, _edges, 0)
        return carry

    lax.fori_loop(c0, c1, _chunk, 0)

    pltpu.sync_copy(acc_v, out_hbm.at[pl.ds(wid * ACCLEN, ACCLEN)])


_sc_call = functools.partial(
    pl.kernel,
    out_type=jax.ShapeDtypeStruct((OUTLEN,), _f32),
    mesh=plsc.VectorSubcoreMesh(core_axis_name="c", subcore_axis_name="s"),
    compiler_params=pltpu.CompilerParams(
        needs_layout_passes=False, use_tc_tiling_on_sc=False),
    scratch_types=[
        pltpu.VMEM((ACCLEN,), _f32),
        pltpu.VMEM((CH, FW), _f32),
        pltpu.VMEM((CH,), _i32),
        pltpu.VMEM((CH,), _i32),
        pltpu.VMEM((CH,), _f32),
        pltpu.VMEM((NPW * H,), _f32),
        pltpu.VMEM((H * CH,), _f32),
        pltpu.VMEM((CH,), _i32),
        pltpu.VMEM((2 * NW,), _i32),
        pltpu.SemaphoreType.DMA,
    ],
)(_sc_body)


BN = 2000


def _post_body(acc_ref, fs_ref, b_ref, bt_ref, scale_ref, offset_ref,
               out_ref):
    acc = acc_ref[...]
    fs = fs_ref[...]
    b = b_ref[...]
    bt = bt_ref[...]
    dn = jnp.clip(acc[:, DIM:DIM + H], 1e-10, None)
    neigh = acc[:, :DIM] / _dot(dn, bt)

    def _norm(f, i):
        mh = _dot(f, b) * (1.0 / D)
        mean = _dot(mh, bt)
        sq = _dot(f * f, b) * (1.0 / D)
        var = _dot(sq, bt) - mean * mean + 1e-9
        return (f - mean) * scale_ref[i] * lax.rsqrt(var) + offset_ref[i]

    out_ref[...] = (_norm(neigh, 0) + _norm(fs, 1)) * 0.5


_post_call = pl.pallas_call(
    _post_body,
    grid=(N // BN,),
    in_specs=[
        pl.BlockSpec((BN, ACCW), lambda i: (i, 0)),
        pl.BlockSpec((BN, DIM), lambda i: (i, 0)),
        pl.BlockSpec((DIM, H), lambda i: (0, 0)),
        pl.BlockSpec((H, DIM), lambda i: (0, 0)),
        pl.BlockSpec((2, DIM), lambda i: (0, 0)),
        pl.BlockSpec((2, DIM), lambda i: (0, 0)),
    ],
    out_specs=pl.BlockSpec((BN, DIM), lambda i: (i, 0)),
    out_shape=jax.ShapeDtypeStruct((N, DIM), _f32),
)


def kernel(x, adj_row, adj_col, adj_val, sizes_subg, W0, b0, W1, b1,
           attention, scale, offset):
    del sizes_subg
    hb = jnp.repeat(jnp.arange(H, dtype=_i32), D)
    onehot = (hb[:, None] == jnp.arange(H, dtype=_i32)[None, :]).astype(_f32)
    a0 = onehot * attention[0].reshape(H * D)[:, None]
    a1 = onehot * attention[1].reshape(H * D)[:, None]

    fs, fnext, eas = _pre_call(x, W0.T, b0[None, :], W1.T, b1[None, :],
                               a0, a1)

    boundaries = jnp.minimum(jnp.arange(NW + 1, dtype=_i32) * NPW, N)
    s = jnp.searchsorted(adj_row, boundaries, side="left").astype(_i32)
    c0 = s[:NW] // CH
    c1 = (s[1:] + CH - 1) // CH
    bounds = jnp.concatenate([c0, c1]).astype(_i32)

    acc = _sc_call(fnext, eas.reshape(-1), adj_row, adj_col, adj_val, bounds)
    acc2d = acc.reshape(NW * NPW, ACCW)

    scale128 = scale.reshape(2, DIM)
    offset128 = offset.reshape(2, DIM)
    out = _post_call(acc2d[:N], fs, onehot, onehot.T, scale128, offset128)
    return out
